# Initial kernel scaffold; baseline (speedup 1.0000x reference)
import functools

import jax
import jax.numpy as jnp
from jax import lax
from jax.experimental import pallas as pl
from jax.experimental.pallas import tpu as pltpu
from jax.experimental.pallas import tpu_sc as plsc

_N = 10000
_E = 320000
_D_IN = 128
_HID = 128
_OUT = 64

_NC = 2
_NS = 16
_NW = _NC * _NS
_PER_W = _E // _NW
_CH = 80
_NCHUNK = _PER_W // _CH
_RPT = _N // _NS
_WCH = 125
_DEGW = 16

_BR = 2000
_GRID = _N // _BR

_HIGH = lax.Precision.HIGHEST


def _dot(a, b):
    return lax.dot_general(a, b, (((1,), (0,)), ((), ())),
                           precision=_HIGH, preferred_element_type=jnp.float32)


def _make_sc_agg(d, with_deg):
    mesh = plsc.VectorSubcoreMesh(core_axis_name="c", subcore_axis_name="s")
    out_type = [jax.ShapeDtypeStruct((_NC, _N, d), jnp.float32)]
    scratch = [
        pltpu.VMEM((_NCHUNK, _CH), jnp.int32),
        pltpu.VMEM((_NCHUNK, _CH), jnp.int32),
        pltpu.VMEM((_CH, d), jnp.float32),
        pltpu.VMEM((_WCH, d), jnp.float32),
        pltpu.VMEM_SHARED((_N, d), jnp.float32),
    ]
    if with_deg:
        out_type.append(jax.ShapeDtypeStruct((_NC, _N, _DEGW), jnp.float32))
        scratch += [
            pltpu.VMEM((_CH, _DEGW), jnp.float32),
            pltpu.VMEM((_WCH, _DEGW), jnp.float32),
            pltpu.VMEM_SHARED((_N, _DEGW), jnp.float32),
        ]

    if with_deg:
        @functools.partial(pl.kernel, mesh=mesh, out_type=out_type,
                           scratch_types=scratch)
        def k(feat_hbm, src_hbm, dst_hbm, zf_hbm, od_hbm, zd_hbm,
              acc_out, deg_out,
              srcv, dstv, rows, wbuf, acc_sh, onesv, dwbuf, deg_sh):
            c = lax.axis_index("c")
            s = lax.axis_index("s")
            w = c * _NS + s
            base = s * _RPT
            pltpu.sync_copy(src_hbm.at[w], srcv)
            pltpu.sync_copy(dst_hbm.at[w], dstv)
            pltpu.sync_copy(zf_hbm, wbuf)
            pltpu.sync_copy(od_hbm, onesv)
            pltpu.sync_copy(zd_hbm, dwbuf)

            @pl.loop(0, _RPT, step=_WCH)
            def _(r):
                pltpu.sync_copy(wbuf, acc_sh.at[pl.ds(base + r, _WCH)])
                pltpu.sync_copy(dwbuf, deg_sh.at[pl.ds(base + r, _WCH)])

            plsc.subcore_barrier()

            @pl.loop(0, _NCHUNK)
            def _(j):
                pltpu.sync_copy(feat_hbm.at[srcv.at[j]], rows)
                pltpu.sync_copy(rows, acc_sh.at[dstv.at[j]], add=True)
                pltpu.sync_copy(onesv, deg_sh.at[dstv.at[j]], add=True)

            plsc.subcore_barrier()

            @pl.loop(0, _RPT, step=_WCH)
            def _(r):
                pltpu.sync_copy(acc_sh.at[pl.ds(base + r, _WCH)], wbuf)
                pltpu.sync_copy(wbuf, acc_out.at[c, pl.ds(base + r, _WCH)])
                pltpu.sync_copy(deg_sh.at[pl.ds(base + r, _WCH)], dwbuf)
                pltpu.sync_copy(dwbuf, deg_out.at[c, pl.ds(base + r, _WCH)])
        return k

    @functools.partial(pl.kernel, mesh=mesh, out_type=out_type,
                       scratch_types=scratch)
    def k2(feat_hbm, src_hbm, dst_hbm, zf_hbm,
           acc_out,
           srcv, dstv, rows, wbuf, acc_sh):
        c = lax.axis_index("c")
        s = lax.axis_index("s")
        w = c * _NS + s
        base = s * _RPT
        pltpu.sync_copy(src_hbm.at[w], srcv)
        pltpu.sync_copy(dst_hbm.at[w], dstv)
        pltpu.sync_copy(zf_hbm, wbuf)

        @pl.loop(0, _RPT, step=_WCH)
        def _(r):
            pltpu.sync_copy(wbuf, acc_sh.at[pl.ds(base + r, _WCH)])

        plsc.subcore_barrier()

        @pl.loop(0, _NCHUNK)
        def _(j):
            pltpu.sync_copy(feat_hbm.at[srcv.at[j]], rows)
            pltpu.sync_copy(rows, acc_sh.at[dstv.at[j]], add=True)

        plsc.subcore_barrier()

        @pl.loop(0, _RPT, step=_WCH)
        def _(r):
            pltpu.sync_copy(acc_sh.at[pl.ds(base + r, _WCH)], wbuf)
            pltpu.sync_copy(wbuf, acc_out.at[c, pl.ds(base + r, _WCH)])
    return k2


_sc_agg_deg = _make_sc_agg(_HID, True)
_sc_agg2 = _make_sc_agg(_OUT, False)


def _in_mm_body(x_ref, w1l_ref, w1r_ref, xl_ref, xr_ref):
    xv = x_ref[...]
    xl_ref[...] = _dot(xv, w1l_ref[...])
    xr_ref[...] = _dot(xv, w1r_ref[...])


def _mid_body(p_ref, dp_ref, xr_ref, b1_ref, w2l_ref, w2r_ref,
              hl_ref, hr_ref):
    dpv = dp_ref[...]
    rdeg = 1.0 / jnp.maximum(dpv[0, :, 0:1] + dpv[1, :, 0:1], 1.0)
    pv = p_ref[...]
    mean = (pv[0] + pv[1]) * rdeg
    h = jnp.maximum(mean + b1_ref[...] + xr_ref[...], 0.0)
    hl_ref[...] = _dot(h, w2l_ref[...])
    hr_ref[...] = _dot(h, w2r_ref[...])


def _final_body(q_ref, dp_ref, hr_ref, b2_ref, o_ref):
    dpv = dp_ref[...]
    rdeg = 1.0 / jnp.maximum(dpv[0, :, 0:1] + dpv[1, :, 0:1], 1.0)
    qv = q_ref[...]
    o_ref[...] = (qv[0] + qv[1]) * rdeg + b2_ref[...] + hr_ref[...]


def _in_mm(x, W1l, W1r):
    return pl.pallas_call(
        _in_mm_body,
        grid=(_GRID,),
        in_specs=[
            pl.BlockSpec((_BR, _D_IN), lambda i: (i, 0)),
            pl.BlockSpec((_D_IN, _HID), lambda i: (0, 0)),
            pl.BlockSpec((_D_IN, _HID), lambda i: (0, 0)),
        ],
        out_specs=[
            pl.BlockSpec((_BR, _HID), lambda i: (i, 0)),
            pl.BlockSpec((_BR, _HID), lambda i: (i, 0)),
        ],
        out_shape=[
            jax.ShapeDtypeStruct((_N, _HID), jnp.float32),
            jax.ShapeDtypeStruct((_N, _HID), jnp.float32),
        ],
    )(x, W1l, W1r)


def _mid(p, dp, xr, b1, W2l, W2r):
    return pl.pallas_call(
        _mid_body,
        grid=(_GRID,),
        in_specs=[
            pl.BlockSpec((_NC, _BR, _HID), lambda i: (0, i, 0)),
            pl.BlockSpec((_NC, _BR, _DEGW), lambda i: (0, i, 0)),
            pl.BlockSpec((_BR, _HID), lambda i: (i, 0)),
            pl.BlockSpec((1, _HID), lambda i: (0, 0)),
            pl.BlockSpec((_HID, _OUT), lambda i: (0, 0)),
            pl.BlockSpec((_HID, _OUT), lambda i: (0, 0)),
        ],
        out_specs=[
            pl.BlockSpec((_BR, _OUT), lambda i: (i, 0)),
            pl.BlockSpec((_BR, _OUT), lambda i: (i, 0)),
        ],
        out_shape=[
            jax.ShapeDtypeStruct((_N, _OUT), jnp.float32),
            jax.ShapeDtypeStruct((_N, _OUT), jnp.float32),
        ],
    )(p, dp, xr, b1, W2l, W2r)


def _final(q, dp, hr, b2):
    return pl.pallas_call(
        _final_body,
        grid=(_GRID,),
        in_specs=[
            pl.BlockSpec((_NC, _BR, _OUT), lambda i: (0, i, 0)),
            pl.BlockSpec((_NC, _BR, _DEGW), lambda i: (0, i, 0)),
            pl.BlockSpec((_BR, _OUT), lambda i: (i, 0)),
            pl.BlockSpec((1, _OUT), lambda i: (0, 0)),
        ],
        out_specs=pl.BlockSpec((_BR, _OUT), lambda i: (i, 0)),
        out_shape=jax.ShapeDtypeStruct((_N, _OUT), jnp.float32),
    )(q, dp, hr, b2)


def kernel(x, edge_index, W1l, b1, W1r, W2l, b2, W2r):
    src_w = edge_index[0].reshape(_NW, _NCHUNK, _CH)
    dst_w = edge_index[1].reshape(_NW, _NCHUNK, _CH)

    zf1 = jnp.zeros((_WCH, _HID), jnp.float32)
    zf2 = jnp.zeros((_WCH, _OUT), jnp.float32)
    od = jnp.ones((_CH, _DEGW), jnp.float32)
    zd = jnp.zeros((_WCH, _DEGW), jnp.float32)

    xl, xr = _in_mm(x, W1l, W1r)
    p, dp = _sc_agg_deg(xl, src_w, dst_w, zf1, od, zd)
    hl, hr = _mid(p, dp, xr, b1.reshape(1, _HID), W2l, W2r)
    q = _sc_agg2(hl, src_w, dst_w, zf2)
    return _final(q, dp, hr, b2.reshape(1, _OUT))


# speedup vs baseline: 8.0302x; 8.0302x over previous
import functools

import jax
import jax.numpy as jnp
from jax import lax
from jax.experimental import pallas as pl
from jax.experimental.pallas import tpu as pltpu
from jax.experimental.pallas import tpu_sc as plsc

_N = 10000
_E = 320000
_D_IN = 128
_HID = 128
_OUT = 64

_NC = 2
_NS = 16
_NW = _NC * _NS
_PER_W = _E // _NW
_CH = 80
_NCHUNK = _PER_W // _CH
_RPT = _N // _NS
_WCH = 25
_DEGW = 16

_BR = 2000
_GRID = _N // _BR

_HIGH = lax.Precision.HIGHEST


def _dot(a, b):
    return lax.dot_general(a, b, (((1,), (0,)), ((), ())),
                           precision=_HIGH, preferred_element_type=jnp.float32)


def _make_sc_agg(d, with_deg):
    mesh = plsc.VectorSubcoreMesh(core_axis_name="c", subcore_axis_name="s")
    cparams = pltpu.CompilerParams(use_tc_tiling_on_sc=False)
    out_type = [jax.ShapeDtypeStruct((_NC, _N, d), jnp.float32)]
    scratch = [
        pltpu.VMEM((_NCHUNK, _CH), jnp.int32),
        pltpu.VMEM((_NCHUNK, _CH), jnp.int32),
        pltpu.VMEM((_CH, d), jnp.float32),
        pltpu.VMEM((_WCH, d), jnp.float32),
        pltpu.VMEM_SHARED((_N, d), jnp.float32),
    ]
    if with_deg:
        out_type.append(jax.ShapeDtypeStruct((_NC, _N, _DEGW), jnp.float32))
        scratch += [
            pltpu.VMEM((_CH, _DEGW), jnp.float32),
            pltpu.VMEM((_WCH, _DEGW), jnp.float32),
            pltpu.VMEM_SHARED((_N, _DEGW), jnp.float32),
        ]

    if with_deg:
        @functools.partial(pl.kernel, mesh=mesh, out_type=out_type,
                           scratch_types=scratch, compiler_params=cparams)
        def k(feat_hbm, src_hbm, dst_hbm, zf_hbm, od_hbm, zd_hbm,
              acc_out, deg_out,
              srcv, dstv, rows, wbuf, acc_sh, onesv, dwbuf, deg_sh):
            c = lax.axis_index("c")
            s = lax.axis_index("s")
            w = c * _NS + s
            base = s * _RPT
            pltpu.sync_copy(src_hbm.at[w], srcv)
            pltpu.sync_copy(dst_hbm.at[w], dstv)
            pltpu.sync_copy(zf_hbm, wbuf)
            pltpu.sync_copy(od_hbm, onesv)
            pltpu.sync_copy(zd_hbm, dwbuf)

            @pl.loop(0, _RPT, step=_WCH)
            def _(r):
                pltpu.sync_copy(wbuf, acc_sh.at[pl.ds(base + r, _WCH)])
                pltpu.sync_copy(dwbuf, deg_sh.at[pl.ds(base + r, _WCH)])

            plsc.subcore_barrier()

            @pl.loop(0, _NCHUNK)
            def _(j):
                pltpu.sync_copy(feat_hbm.at[srcv.at[j]], rows)
                pltpu.sync_copy(rows, acc_sh.at[dstv.at[j]], add=True)
                pltpu.sync_copy(onesv, deg_sh.at[dstv.at[j]], add=True)

            plsc.subcore_barrier()

            @pl.loop(0, _RPT, step=_WCH)
            def _(r):
                pltpu.sync_copy(acc_sh.at[pl.ds(base + r, _WCH)], wbuf)
                pltpu.sync_copy(wbuf, acc_out.at[c, pl.ds(base + r, _WCH)])
                pltpu.sync_copy(deg_sh.at[pl.ds(base + r, _WCH)], dwbuf)
                pltpu.sync_copy(dwbuf, deg_out.at[c, pl.ds(base + r, _WCH)])
        return k

    @functools.partial(pl.kernel, mesh=mesh, out_type=out_type[0],
                       scratch_types=scratch, compiler_params=cparams)
    def k2(feat_hbm, src_hbm, dst_hbm, zf_hbm,
           acc_out,
           srcv, dstv, rows, wbuf, acc_sh):
        c = lax.axis_index("c")
        s = lax.axis_index("s")
        w = c * _NS + s
        base = s * _RPT
        pltpu.sync_copy(src_hbm.at[w], srcv)
        pltpu.sync_copy(dst_hbm.at[w], dstv)
        pltpu.sync_copy(zf_hbm, wbuf)

        @pl.loop(0, _RPT, step=_WCH)
        def _(r):
            pltpu.sync_copy(wbuf, acc_sh.at[pl.ds(base + r, _WCH)])

        plsc.subcore_barrier()

        @pl.loop(0, _NCHUNK)
        def _(j):
            pltpu.sync_copy(feat_hbm.at[srcv.at[j]], rows)
            pltpu.sync_copy(rows, acc_sh.at[dstv.at[j]], add=True)

        plsc.subcore_barrier()

        @pl.loop(0, _RPT, step=_WCH)
        def _(r):
            pltpu.sync_copy(acc_sh.at[pl.ds(base + r, _WCH)], wbuf)
            pltpu.sync_copy(wbuf, acc_out.at[c, pl.ds(base + r, _WCH)])
    return k2


_sc_agg_deg = _make_sc_agg(_HID, True)
_sc_agg2 = _make_sc_agg(_OUT, False)


def _in_mm_body(x_ref, w1l_ref, w1r_ref, xl_ref, xr_ref):
    xv = x_ref[...]
    xl_ref[...] = _dot(xv, w1l_ref[...])
    xr_ref[...] = _dot(xv, w1r_ref[...])


def _mid_body(p_ref, dp_ref, xr_ref, b1_ref, w2l_ref, w2r_ref,
              hl_ref, hr_ref):
    dpv = dp_ref[...]
    rdeg = 1.0 / jnp.maximum(dpv[0, :, 0:1] + dpv[1, :, 0:1], 1.0)
    pv = p_ref[...]
    mean = (pv[0] + pv[1]) * rdeg
    h = jnp.maximum(mean + b1_ref[...] + xr_ref[...], 0.0)
    hl_ref[...] = _dot(h, w2l_ref[...])
    hr_ref[...] = _dot(h, w2r_ref[...])


def _final_body(q_ref, dp_ref, hr_ref, b2_ref, o_ref):
    dpv = dp_ref[...]
    rdeg = 1.0 / jnp.maximum(dpv[0, :, 0:1] + dpv[1, :, 0:1], 1.0)
    qv = q_ref[...]
    o_ref[...] = (qv[0] + qv[1]) * rdeg + b2_ref[...] + hr_ref[...]


def _in_mm(x, W1l, W1r):
    return pl.pallas_call(
        _in_mm_body,
        grid=(_GRID,),
        in_specs=[
            pl.BlockSpec((_BR, _D_IN), lambda i: (i, 0)),
            pl.BlockSpec((_D_IN, _HID), lambda i: (0, 0)),
            pl.BlockSpec((_D_IN, _HID), lambda i: (0, 0)),
        ],
        out_specs=[
            pl.BlockSpec((_BR, _HID), lambda i: (i, 0)),
            pl.BlockSpec((_BR, _HID), lambda i: (i, 0)),
        ],
        out_shape=[
            jax.ShapeDtypeStruct((_N, _HID), jnp.float32),
            jax.ShapeDtypeStruct((_N, _HID), jnp.float32),
        ],
    )(x, W1l, W1r)


def _mid(p, dp, xr, b1, W2l, W2r):
    return pl.pallas_call(
        _mid_body,
        grid=(_GRID,),
        in_specs=[
            pl.BlockSpec((_NC, _BR, _HID), lambda i: (0, i, 0)),
            pl.BlockSpec((_NC, _BR, _DEGW), lambda i: (0, i, 0)),
            pl.BlockSpec((_BR, _HID), lambda i: (i, 0)),
            pl.BlockSpec((1, _HID), lambda i: (0, 0)),
            pl.BlockSpec((_HID, _OUT), lambda i: (0, 0)),
            pl.BlockSpec((_HID, _OUT), lambda i: (0, 0)),
        ],
        out_specs=[
            pl.BlockSpec((_BR, _OUT), lambda i: (i, 0)),
            pl.BlockSpec((_BR, _OUT), lambda i: (i, 0)),
        ],
        out_shape=[
            jax.ShapeDtypeStruct((_N, _OUT), jnp.float32),
            jax.ShapeDtypeStruct((_N, _OUT), jnp.float32),
        ],
    )(p, dp, xr, b1, W2l, W2r)


def _final(q, dp, hr, b2):
    return pl.pallas_call(
        _final_body,
        grid=(_GRID,),
        in_specs=[
            pl.BlockSpec((_NC, _BR, _OUT), lambda i: (0, i, 0)),
            pl.BlockSpec((_NC, _BR, _DEGW), lambda i: (0, i, 0)),
            pl.BlockSpec((_BR, _OUT), lambda i: (i, 0)),
            pl.BlockSpec((1, _OUT), lambda i: (0, 0)),
        ],
        out_specs=pl.BlockSpec((_BR, _OUT), lambda i: (i, 0)),
        out_shape=jax.ShapeDtypeStruct((_N, _OUT), jnp.float32),
    )(q, dp, hr, b2)


def kernel(x, edge_index, W1l, b1, W1r, W2l, b2, W2r):
    src_w = edge_index[0].reshape(_NW, _NCHUNK, _CH)
    dst_w = edge_index[1].reshape(_NW, _NCHUNK, _CH)

    zf1 = jnp.zeros((_WCH, _HID), jnp.float32)
    zf2 = jnp.zeros((_WCH, _OUT), jnp.float32)
    od = jnp.ones((_CH, _DEGW), jnp.float32)
    zd = jnp.zeros((_WCH, _DEGW), jnp.float32)

    xl, xr = _in_mm(x, W1l, W1r)
    p, dp = _sc_agg_deg(xl, src_w, dst_w, zf1, od, zd)
    hl, hr = _mid(p, dp, xr, b1.reshape(1, _HID), W2l, W2r)
    q = _sc_agg2(hl, src_w, dst_w, zf2)
    return _final(q, dp, hr, b2.reshape(1, _OUT))
